# baseline (speedup 1.0000x reference)
import functools
from math import ceil

import jax
import jax.numpy as jnp
from jax import lax
from jax.experimental import pallas as pl
from jax.experimental.pallas import tpu as pltpu
from jax.experimental.pallas import tpu_sc as plsc

N = 10000
E = 320000
D = 128
RATIO = 0.5
NEG = -jnp.inf

NC = 2
NS = 16
NW = NC * NS
CHUNK = 128
NCHUNK = 80
EPAD = NW * NCHUNK * CHUNK
EPW = E // NW
NPAD = 10240
RPT = NPAD // NS
ZR = 128


@functools.partial(
    pl.kernel,
    out_type=jax.ShapeDtypeStruct((NC, NPAD, D), jnp.float32),
    mesh=plsc.VectorSubcoreMesh(core_axis_name="c", subcore_axis_name="s"),
    scratch_types=[
        pltpu.VMEM((NCHUNK, CHUNK), jnp.int32),
        pltpu.VMEM((NCHUNK, CHUNK), jnp.int32),
        pltpu.VMEM((CHUNK, D), jnp.float32),
        pltpu.VMEM_SHARED((NPAD, D), jnp.float32),
        pltpu.SemaphoreType.DMA,
    ],
)
def _msg_kernel(g_hbm, src_hbm, dst_hbm, zero_hbm, out_hbm,
                idx_s, idx_d, buf_a, acc, sem_ra):
    c = lax.axis_index("c")
    s = lax.axis_index("s")
    wid = s * NC + c

    pltpu.sync_copy(src_hbm.at[wid], idx_s)
    pltpu.sync_copy(dst_hbm.at[wid], idx_d)

    pltpu.sync_copy(zero_hbm, buf_a.at[pl.ds(0, ZR)])
    for j in range(RPT // ZR):
        pltpu.sync_copy(buf_a.at[pl.ds(0, ZR)], acc.at[pl.ds(s * RPT + j * ZR, ZR)])
    plsc.subcore_barrier()

    def body(i, carry):
        pltpu.async_copy(g_hbm.at[idx_s.at[i]], buf_a, sem_ra).wait()
        pltpu.sync_copy(buf_a, acc.at[idx_d.at[i]], add=True)
        return carry

    lax.fori_loop(0, NCHUNK, body, 0)
    plsc.subcore_barrier()

    for j in range(RPT // ZR):
        r = s * RPT + j * ZR
        pltpu.sync_copy(acc.at[pl.ds(r, ZR)], buf_a.at[pl.ds(0, ZR)])
        pltpu.sync_copy(buf_a.at[pl.ds(0, ZR)], out_hbm.at[c, pl.ds(r, ZR)])


def _msg_segsum(gpad, src3, dst3, zeros):
    parts = _msg_kernel(gpad, src3, dst3, zeros)
    return (parts[0] + parts[1])[:N]


@functools.partial(
    pl.kernel,
    out_type=jax.ShapeDtypeStruct((NW, N), jnp.float32),
    mesh=plsc.VectorSubcoreMesh(core_axis_name="c", subcore_axis_name="s"),
    scratch_types=[
        pltpu.VMEM((N,), jnp.float32),
        pltpu.VMEM((EPW,), jnp.int32),
        pltpu.VMEM((EPW,), jnp.int32),
        pltpu.VMEM((N,), jnp.float32),
    ],
    compiler_params=pltpu.CompilerParams(needs_layout_passes=False),
)
def _scalar_kernel(sn_hbm, src_hbm, dst_hbm, out_hbm, sn_v, src_v, dst_v, acc_v):
    c = lax.axis_index("c")
    s = lax.axis_index("s")
    wid = s * NC + c
    pltpu.sync_copy(sn_hbm, sn_v)
    pltpu.sync_copy(src_hbm.at[pl.ds(wid * EPW, EPW)], src_v)
    pltpu.sync_copy(dst_hbm.at[pl.ds(wid * EPW, EPW)], dst_v)
    zv = jnp.zeros((16,), jnp.float32)

    def zbody(i, carry):
        acc_v[pl.ds(i * 16, 16)] = zv
        return carry

    lax.fori_loop(0, N // 16, zbody, 0)

    def body(i, carry):
        sidx = src_v[pl.ds(i * 16, 16)]
        v = plsc.load_gather(sn_v, [sidx])
        didx = dst_v[pl.ds(i * 16, 16)]
        plsc.addupdate_scatter(acc_v, [didx], v)
        return carry

    lax.fori_loop(0, EPW // 16, body, 0)
    pltpu.sync_copy(acc_v, out_hbm.at[wid])


def _scalar_segsum(sn, src, dst):
    return jnp.sum(_scalar_kernel(sn, src, dst), axis=0)


def _mlp_body(z_ref, w1_ref, b1_ref, w2_ref, b2_ref, w3_ref, b3_ref, out_ref):
    z = z_ref[...]
    z = jax.nn.relu(z @ w1_ref[...] + b1_ref[...][None, :])
    z = jax.nn.relu(z @ w2_ref[...] + b2_ref[...][None, :])
    z = z @ w3_ref[...] + b3_ref[...][None, :]
    m = jnp.max(z, axis=-1, keepdims=True)
    e = jnp.exp(z - m)
    out_ref[...] = z - m - jnp.log(jnp.sum(e, axis=-1, keepdims=True))


def _mlp(z, w1, b1, w2, b2, w3, b3):
    return pl.pallas_call(
        _mlp_body,
        out_shape=jax.ShapeDtypeStruct((1, w3.shape[1]), jnp.float32),
    )(z, w1, b1, w2, b2, w3, b3)


def kernel(x, c1_wr, c1_wn, c1_b, p1_wr, p1_wn, p1_b,
           c2_wr, c2_wn, c2_b, p2_wr, p2_wn, p2_b,
           c3_wr, c3_wn, c3_b, p3_wr, p3_wn, p3_b,
           l1_w, l1_b, l2_w, l2_b, l3_w, l3_b,
           edge_index, two_hop, batch):
    src, dst = edge_index[0], edge_index[1]
    zeros = jnp.zeros((ZR, D), jnp.float32)
    pad = jnp.full((EPAD - E,), N, jnp.int32)
    src3 = jnp.concatenate([src, pad]).reshape(NW, NCHUNK, CHUNK)
    dst3 = jnp.concatenate([dst, pad]).reshape(NW, NCHUNK, CHUNK)
    zpad = jnp.zeros((NPAD - N, D), jnp.float32)
    n = x.shape[0]
    mask = jnp.ones((n,), jnp.float32)
    hr = x @ c1_wr
    g = x @ c1_wn
    z = jnp.zeros((1, 2 * D), jnp.float32)
    k_prev = n
    layers = [(c1_b, p1_wr, p1_wn, p1_b, c2_wn, c2_wr),
              (c2_b, p2_wr, p2_wn, p2_b, c3_wn, c3_wr),
              (c3_b, p3_wr, p3_wn, p3_b, None, None)]
    for (cb, pwr, pwn, pb, wn_next, wr_next) in layers:
        agg = _msg_segsum(jnp.concatenate([g, zpad]), src3, dst3, zeros)
        h = jax.nn.relu(hr + agg + cb) * mask[:, None]
        sr = (h @ pwr)[:, 0]
        sn = (h @ pwn)[:, 0]
        sagg = _scalar_segsum(sn, src, dst)
        s = sr + sagg + pb[0]
        smask = jnp.where(mask > 0, s, NEG)
        k = int(ceil(RATIO * k_prev))
        thr = jnp.sort(smask)[n - k]
        mask_new = (smask >= thr).astype(jnp.float32)
        t = jnp.tanh(s) * mask_new
        xn = h * t[:, None]
        rmax = jnp.max(jnp.where(mask_new[:, None] > 0, xn, NEG), axis=0)
        rmean = jnp.sum(xn, axis=0) / k
        z = z + jnp.concatenate([rmax, rmean])[None, :]
        if wn_next is not None:
            g = (h @ wn_next) * t[:, None]
            hr = (h @ wr_next) * t[:, None]
        mask = mask_new
        k_prev = k
    return _mlp(z, l1_w, l1_b, l2_w, l2_b, l3_w, l3_b)


# speedup vs baseline: 1.5725x; 1.5725x over previous
import functools
from math import ceil

import jax
import jax.numpy as jnp
from jax import lax
from jax.experimental import pallas as pl
from jax.experimental.pallas import tpu as pltpu
from jax.experimental.pallas import tpu_sc as plsc

N = 10000
E = 320000
D = 128
RATIO = 0.5
NEG = -jnp.inf

NC = 2
NS = 16
NW = NC * NS
CHUNK = 80
NCHUNK = E // NW // CHUNK
EPW = E // NW
NPAD = 10240
RPT = NPAD // NS
ZR = 128


@functools.partial(
    pl.kernel,
    out_type=jax.ShapeDtypeStruct((NC, NPAD, D), jnp.float32),
    mesh=plsc.VectorSubcoreMesh(core_axis_name="c", subcore_axis_name="s"),
    scratch_types=[
        pltpu.VMEM((CHUNK,), jnp.int32),
        pltpu.VMEM((CHUNK,), jnp.int32),
        pltpu.VMEM((CHUNK, D), jnp.float32),
        pltpu.VMEM((ZR, D), jnp.float32),
        pltpu.VMEM_SHARED((NPAD, D), jnp.float32),
        pltpu.SemaphoreType.DMA,
    ],
)
def _msg_kernel(g_hbm, src_hbm, dst_hbm, zero_hbm, out_hbm,
                idx_s, idx_d, rows, bounce, acc, sem):
    c = lax.axis_index("c")
    s = lax.axis_index("s")
    wid = s * NC + c

    pltpu.sync_copy(zero_hbm, bounce)
    for j in range(RPT // ZR):
        pltpu.sync_copy(bounce, acc.at[pl.ds(s * RPT + j * ZR, ZR)])
    plsc.subcore_barrier()

    def body(i, carry):
        base = wid * EPW + i * CHUNK
        pltpu.sync_copy(src_hbm.at[pl.ds(base, CHUNK)], idx_s)
        pltpu.sync_copy(dst_hbm.at[pl.ds(base, CHUNK)], idx_d)
        pltpu.async_copy(g_hbm.at[idx_s], rows, sem).wait()
        pltpu.sync_copy(rows, acc.at[idx_d], add=True)
        return carry

    lax.fori_loop(0, NCHUNK, body, 0)
    plsc.subcore_barrier()

    for j in range(RPT // ZR):
        r = s * RPT + j * ZR
        pltpu.sync_copy(acc.at[pl.ds(r, ZR)], bounce)
        pltpu.sync_copy(bounce, out_hbm.at[c, pl.ds(r, ZR)])


def _msg_segsum(g, src, dst, zeros):
    parts = _msg_kernel(g, src, dst, zeros)
    return (parts[0] + parts[1])[:N]


@functools.partial(
    pl.kernel,
    out_type=jax.ShapeDtypeStruct((NW, N), jnp.float32),
    mesh=plsc.VectorSubcoreMesh(core_axis_name="c", subcore_axis_name="s"),
    scratch_types=[
        pltpu.VMEM((N,), jnp.float32),
        pltpu.VMEM((EPW,), jnp.int32),
        pltpu.VMEM((EPW,), jnp.int32),
        pltpu.VMEM((N,), jnp.float32),
    ],
    compiler_params=pltpu.CompilerParams(needs_layout_passes=False),
)
def _scalar_kernel(sn_hbm, src_hbm, dst_hbm, out_hbm, sn_v, src_v, dst_v, acc_v):
    c = lax.axis_index("c")
    s = lax.axis_index("s")
    wid = s * NC + c
    pltpu.sync_copy(sn_hbm, sn_v)
    pltpu.sync_copy(src_hbm.at[pl.ds(wid * EPW, EPW)], src_v)
    pltpu.sync_copy(dst_hbm.at[pl.ds(wid * EPW, EPW)], dst_v)
    zv = jnp.zeros((16,), jnp.float32)

    def zbody(i, carry):
        acc_v[pl.ds(i * 16, 16)] = zv
        return carry

    lax.fori_loop(0, N // 16, zbody, 0)

    def body(i, carry):
        sidx = src_v[pl.ds(i * 16, 16)]
        v = plsc.load_gather(sn_v, [sidx])
        didx = dst_v[pl.ds(i * 16, 16)]
        plsc.addupdate_scatter(acc_v, [didx], v)
        return carry

    lax.fori_loop(0, EPW // 16, body, 0)
    pltpu.sync_copy(acc_v, out_hbm.at[wid])


def _scalar_segsum(sn, src, dst):
    return jnp.sum(_scalar_kernel(sn, src, dst), axis=0)


def _mlp_body(z_ref, w1_ref, b1_ref, w2_ref, b2_ref, w3_ref, b3_ref, out_ref):
    z = z_ref[...]
    z = jax.nn.relu(z @ w1_ref[...] + b1_ref[...][None, :])
    z = jax.nn.relu(z @ w2_ref[...] + b2_ref[...][None, :])
    z = z @ w3_ref[...] + b3_ref[...][None, :]
    m = jnp.max(z, axis=-1, keepdims=True)
    e = jnp.exp(z - m)
    out_ref[...] = z - m - jnp.log(jnp.sum(e, axis=-1, keepdims=True))


def _mlp(z, w1, b1, w2, b2, w3, b3):
    return pl.pallas_call(
        _mlp_body,
        out_shape=jax.ShapeDtypeStruct((1, w3.shape[1]), jnp.float32),
    )(z, w1, b1, w2, b2, w3, b3)


def kernel(x, c1_wr, c1_wn, c1_b, p1_wr, p1_wn, p1_b,
           c2_wr, c2_wn, c2_b, p2_wr, p2_wn, p2_b,
           c3_wr, c3_wn, c3_b, p3_wr, p3_wn, p3_b,
           l1_w, l1_b, l2_w, l2_b, l3_w, l3_b,
           edge_index, two_hop, batch):
    src, dst = edge_index[0], edge_index[1]
    zeros = jnp.zeros((ZR, D), jnp.float32)
    n = x.shape[0]
    mask = jnp.ones((n,), jnp.float32)
    hr = x @ c1_wr
    g = x @ c1_wn
    z = jnp.zeros((1, 2 * D), jnp.float32)
    k_prev = n
    layers = [(c1_b, p1_wr, p1_wn, p1_b, c2_wn, c2_wr),
              (c2_b, p2_wr, p2_wn, p2_b, c3_wn, c3_wr),
              (c3_b, p3_wr, p3_wn, p3_b, None, None)]
    for (cb, pwr, pwn, pb, wn_next, wr_next) in layers:
        agg = _msg_segsum(g, src, dst, zeros)
        h = jax.nn.relu(hr + agg + cb) * mask[:, None]
        sr = (h @ pwr)[:, 0]
        sn = (h @ pwn)[:, 0]
        sagg = _scalar_segsum(sn, src, dst)
        s = sr + sagg + pb[0]
        smask = jnp.where(mask > 0, s, NEG)
        k = int(ceil(RATIO * k_prev))
        thr = jnp.sort(smask)[n - k]
        mask_new = (smask >= thr).astype(jnp.float32)
        t = jnp.tanh(s) * mask_new
        xn = h * t[:, None]
        rmax = jnp.max(jnp.where(mask_new[:, None] > 0, xn, NEG), axis=0)
        rmean = jnp.sum(xn, axis=0) / k
        z = z + jnp.concatenate([rmax, rmean])[None, :]
        if wn_next is not None:
            g = (h @ wn_next) * t[:, None]
            hr = (h @ wr_next) * t[:, None]
        mask = mask_new
        k_prev = k
    return _mlp(z, l1_w, l1_b, l2_w, l2_b, l3_w, l3_b)


# speedup vs baseline: 2.4605x; 1.5647x over previous
import functools
from math import ceil

import jax
import jax.numpy as jnp
from jax import lax
from jax.experimental import pallas as pl
from jax.experimental.pallas import tpu as pltpu
from jax.experimental.pallas import tpu_sc as plsc

N = 10000
E = 320000
D = 128
RATIO = 0.5
NEG = -jnp.inf

NC = 2
NS = 16
NW = NC * NS
CHUNK = 80
NCHUNK = E // NW // CHUNK
EPW = E // NW
NPAD = 10240
RPT = NPAD // NS
ZR = 128


@functools.partial(
    pl.kernel,
    out_type=jax.ShapeDtypeStruct((NC, NPAD, D), jnp.float32),
    mesh=plsc.VectorSubcoreMesh(core_axis_name="c", subcore_axis_name="s"),
    scratch_types=[
        pltpu.VMEM((CHUNK,), jnp.int32),
        pltpu.VMEM((CHUNK,), jnp.int32),
        pltpu.VMEM((CHUNK, D), jnp.float32),
        pltpu.VMEM((ZR, D), jnp.float32),
        pltpu.VMEM((16,), jnp.int32),
        pltpu.VMEM_SHARED((NPAD, D), jnp.float32),
        pltpu.SemaphoreType.DMA,
    ],
    compiler_params=pltpu.CompilerParams(needs_layout_passes=False),
)
def _msg_kernel(g_hbm, src_hbm, dst_hbm, cnt_hbm, zero_hbm, out_hbm,
                idx_s, idx_d, rows, bounce, cnt_v, acc, sem):
    c = lax.axis_index("c")
    s = lax.axis_index("s")
    wid = s * NC + c
    pltpu.sync_copy(cnt_hbm.at[pl.ds(wid * 16, 16)], cnt_v)

    pltpu.sync_copy(zero_hbm, bounce)
    for j in range(RPT // ZR):
        pltpu.sync_copy(bounce, acc.at[pl.ds(s * RPT + j * ZR, ZR)])
    plsc.subcore_barrier()

    def body(i, carry):
        base = wid * EPW + i * CHUNK
        pltpu.sync_copy(src_hbm.at[pl.ds(base, CHUNK)], idx_s)
        pltpu.sync_copy(dst_hbm.at[pl.ds(base, CHUNK)], idx_d)
        pltpu.async_copy(g_hbm.at[idx_s], rows, sem).wait()
        pltpu.sync_copy(rows, acc.at[idx_d], add=True)
        return carry

    nch = jnp.max(cnt_v[...])
    lax.fori_loop(0, nch, body, 0)
    plsc.subcore_barrier()

    for j in range(RPT // ZR):
        r = s * RPT + j * ZR
        pltpu.sync_copy(acc.at[pl.ds(r, ZR)], bounce)
        pltpu.sync_copy(bounce, out_hbm.at[c, pl.ds(r, ZR)])


def _msg_segsum(g, src, dst, cnt, zeros):
    parts = _msg_kernel(g, src, dst, cnt, zeros)
    return (parts[0] + parts[1])[:N]


@functools.partial(
    pl.kernel,
    out_type=[jax.ShapeDtypeStruct((E,), jnp.int32),
              jax.ShapeDtypeStruct((E,), jnp.int32),
              jax.ShapeDtypeStruct((NW * 16,), jnp.int32)],
    mesh=plsc.VectorSubcoreMesh(core_axis_name="c", subcore_axis_name="s"),
    scratch_types=[
        pltpu.VMEM((N,), jnp.float32),
        pltpu.VMEM((EPW,), jnp.int32),
        pltpu.VMEM((EPW,), jnp.int32),
        pltpu.VMEM((EPW + 96,), jnp.int32),
        pltpu.VMEM((EPW + 96,), jnp.int32),
        pltpu.VMEM((16,), jnp.int32),
    ],
    compiler_params=pltpu.CompilerParams(needs_layout_passes=False),
)
def _compact_kernel(src_hbm, dst_hbm, mask_hbm, csrc_hbm, cdst_hbm, cnt_hbm,
                    mask_v, src_v, dst_v, osrc_v, odst_v, cnt_v):
    c = lax.axis_index("c")
    s = lax.axis_index("s")
    wid = s * NC + c
    base = wid * EPW
    pltpu.sync_copy(mask_hbm, mask_v)
    pltpu.sync_copy(src_hbm.at[pl.ds(base, EPW)], src_v)
    pltpu.sync_copy(dst_hbm.at[pl.ds(base, EPW)], dst_v)

    def body(i, off):
        sv = src_v[pl.ds(i * 16, 16)]
        dv = dst_v[pl.ds(i * 16, 16)]
        ms = plsc.load_gather(mask_v, [sv])
        md = plsc.load_gather(mask_v, [dv])
        keep = (ms * md) > 0.0
        plsc.store_compressed(osrc_v.at[pl.ds(off, 16)], sv, mask=keep)
        plsc.store_compressed(odst_v.at[pl.ds(off, 16)], dv, mask=keep)
        npc = jnp.max(plsc.all_reduce_population_count(keep))
        return off + npc

    off = lax.fori_loop(0, EPW // 16, body, 0)

    zsrc = jnp.zeros((16,), jnp.int32)
    zdst = jnp.full((16,), N, jnp.int32)
    for j in range(6):
        osrc_v[pl.ds(off + j * 16, 16)] = zsrc
        odst_v[pl.ds(off + j * 16, 16)] = zdst

    nch = (off + CHUNK - 1) // CHUNK
    cnt_v[...] = jnp.full((16,), nch, jnp.int32)
    pltpu.sync_copy(cnt_v, cnt_hbm.at[pl.ds(wid * 16, 16)])
    pltpu.sync_copy(osrc_v.at[pl.ds(0, EPW)], csrc_hbm.at[pl.ds(base, EPW)])
    pltpu.sync_copy(odst_v.at[pl.ds(0, EPW)], cdst_hbm.at[pl.ds(base, EPW)])


@functools.partial(
    pl.kernel,
    out_type=jax.ShapeDtypeStruct((NW, NPAD), jnp.float32),
    mesh=plsc.VectorSubcoreMesh(core_axis_name="c", subcore_axis_name="s"),
    scratch_types=[
        pltpu.VMEM((N,), jnp.float32),
        pltpu.VMEM((EPW,), jnp.int32),
        pltpu.VMEM((EPW,), jnp.int32),
        pltpu.VMEM((NPAD,), jnp.float32),
        pltpu.VMEM((16,), jnp.int32),
    ],
    compiler_params=pltpu.CompilerParams(needs_layout_passes=False),
)
def _scalar_kernel(sn_hbm, src_hbm, dst_hbm, cnt_hbm, out_hbm,
                   sn_v, src_v, dst_v, acc_v, cnt_v):
    c = lax.axis_index("c")
    s = lax.axis_index("s")
    wid = s * NC + c
    pltpu.sync_copy(sn_hbm, sn_v)
    pltpu.sync_copy(src_hbm.at[pl.ds(wid * EPW, EPW)], src_v)
    pltpu.sync_copy(dst_hbm.at[pl.ds(wid * EPW, EPW)], dst_v)
    pltpu.sync_copy(cnt_hbm.at[pl.ds(wid * 16, 16)], cnt_v)
    zv = jnp.zeros((16,), jnp.float32)

    def zbody(i, carry):
        acc_v[pl.ds(i * 16, 16)] = zv
        return carry

    lax.fori_loop(0, NPAD // 16, zbody, 0)

    def body(i, carry):
        sidx = src_v[pl.ds(i * 16, 16)]
        v = plsc.load_gather(sn_v, [sidx])
        didx = dst_v[pl.ds(i * 16, 16)]
        plsc.addupdate_scatter(acc_v, [didx], v)
        return carry

    nch = jnp.max(cnt_v[...])
    lax.fori_loop(0, nch * (CHUNK // 16), body, 0)
    pltpu.sync_copy(acc_v, out_hbm.at[wid])


def _scalar_segsum(sn, src, dst, cnt):
    return jnp.sum(_scalar_kernel(sn, src, dst, cnt), axis=0)[:N]


def _mlp_body(z_ref, w1_ref, b1_ref, w2_ref, b2_ref, w3_ref, b3_ref, out_ref):
    z = z_ref[...]
    z = jax.nn.relu(z @ w1_ref[...] + b1_ref[...][None, :])
    z = jax.nn.relu(z @ w2_ref[...] + b2_ref[...][None, :])
    z = z @ w3_ref[...] + b3_ref[...][None, :]
    m = jnp.max(z, axis=-1, keepdims=True)
    e = jnp.exp(z - m)
    out_ref[...] = z - m - jnp.log(jnp.sum(e, axis=-1, keepdims=True))


def _mlp(z, w1, b1, w2, b2, w3, b3):
    return pl.pallas_call(
        _mlp_body,
        out_shape=jax.ShapeDtypeStruct((1, w3.shape[1]), jnp.float32),
    )(z, w1, b1, w2, b2, w3, b3)


def kernel(x, c1_wr, c1_wn, c1_b, p1_wr, p1_wn, p1_b,
           c2_wr, c2_wn, c2_b, p2_wr, p2_wn, p2_b,
           c3_wr, c3_wn, c3_b, p3_wr, p3_wn, p3_b,
           l1_w, l1_b, l2_w, l2_b, l3_w, l3_b,
           edge_index, two_hop, batch):
    src, dst = edge_index[0], edge_index[1]
    zeros = jnp.zeros((ZR, D), jnp.float32)
    cnt = jnp.full((NW * 16,), NCHUNK, jnp.int32)
    n = x.shape[0]
    mask = jnp.ones((n,), jnp.float32)
    hr = x @ c1_wr
    g = x @ c1_wn
    z = jnp.zeros((1, 2 * D), jnp.float32)
    k_prev = n
    layers = [(c1_b, p1_wr, p1_wn, p1_b, c2_wn, c2_wr),
              (c2_b, p2_wr, p2_wn, p2_b, c3_wn, c3_wr),
              (c3_b, p3_wr, p3_wn, p3_b, None, None)]
    for (cb, pwr, pwn, pb, wn_next, wr_next) in layers:
        agg = _msg_segsum(g, src, dst, cnt, zeros)
        h = jax.nn.relu(hr + agg + cb) * mask[:, None]
        sr = (h @ pwr)[:, 0]
        sn = (h @ pwn)[:, 0]
        sagg = _scalar_segsum(sn, src, dst, cnt)
        s = sr + sagg + pb[0]
        smask = jnp.where(mask > 0, s, NEG)
        k = int(ceil(RATIO * k_prev))
        thr = jnp.sort(smask)[n - k]
        mask_new = (smask >= thr).astype(jnp.float32)
        t = jnp.tanh(s) * mask_new
        xn = h * t[:, None]
        rmax = jnp.max(jnp.where(mask_new[:, None] > 0, xn, NEG), axis=0)
        rmean = jnp.sum(xn, axis=0) / k
        z = z + jnp.concatenate([rmax, rmean])[None, :]
        if wn_next is not None:
            g = (h @ wn_next) * t[:, None]
            hr = (h @ wr_next) * t[:, None]
            src, dst, cnt = _compact_kernel(edge_index[0], edge_index[1], mask_new)
        mask = mask_new
        k_prev = k
    return _mlp(z, l1_w, l1_b, l2_w, l2_b, l3_w, l3_b)
